# baseline (speedup 1.0000x reference)
import functools

import jax
import jax.numpy as jnp
from jax.experimental import pallas as pl
from jax.experimental.pallas import tpu as pltpu
from jax.experimental.pallas import tpu_sc as plsc

B, S, H = 8, 512, 768
K_MEM = 65536
KB = 4096
LANES = 16


def _scores_body(h_ref, w1_ref, b1_ref, w2r_ref, b2_ref, mk_ref,
                 fw_ref, idx_ref, q_scr, mv_ref, mt_ref):
    kb = pl.program_id(0)

    @pl.when(kb == 0)
    def _():
        hs = h_ref[...]
        q = jnp.sum(hs, axis=1) * (1.0 / S)
        q_scr[...] = q
        h1 = jnp.maximum(
            jax.lax.dot_general(q, w1_ref[...], (((1,), (0,)), ((), ())),
                                preferred_element_type=jnp.float32)
            + b1_ref[...], 0.0)
        z = jnp.sum(h1 * w2r_ref[...], axis=1, keepdims=True) + b2_ref[...]
        fw_ref[...] = jnp.broadcast_to(jax.nn.sigmoid(z), (B, 128))

    k = mk_ref[...]
    q = q_scr[...]
    qk = jax.lax.dot_general(q, k, (((1,), (1,)), ((), ())),
                             preferred_element_type=jnp.float32)
    kk = k * k
    kkhi = kk.astype(jnp.bfloat16)
    kklo = (kk - kkhi.astype(jnp.float32)).astype(jnp.bfloat16)
    ones_b = jnp.ones((B, H), jnp.bfloat16)
    dn = (((1,), (1,)), ((), ()))
    ksq = (jax.lax.dot_general(ones_b, kkhi, dn,
                               preferred_element_type=jnp.float32)
           + jax.lax.dot_general(ones_b, kklo, dn,
                                 preferred_element_type=jnp.float32))
    s = ksq - 2.0 * qk

    @pl.when(kb == 0)
    def _():
        mv_ref[...] = s
        mt_ref[...] = jnp.zeros((B, KB), jnp.int32)

    @pl.when(kb > 0)
    def _():
        old = mv_ref[...]
        p = s < old
        mt_ref[...] = jnp.where(p, kb, mt_ref[...])
        mv_ref[...] = jnp.where(p, s, old)

    @pl.when(kb == K_MEM // KB - 1)
    def _():
        mv = mv_ref[...]
        gm = mt_ref[...] * KB + jax.lax.broadcasted_iota(
            jnp.int32, (B, KB), 1)
        m = jnp.min(mv, axis=1, keepdims=True)
        sel = jnp.where(mv == m, gm, jnp.int32(2**31 - 1))
        idx8 = jnp.min(sel, axis=1)
        idx_ref[...] = jnp.concatenate(
            [idx8[None, :], jnp.zeros((1, 128 - B), jnp.int32)], axis=1)


def _scores(hidden, g_w1, g_b1, g_w2, g_b2, memory_keys):
    return pl.pallas_call(
        _scores_body,
        grid=(K_MEM // KB,),
        in_specs=[
            pl.BlockSpec((B, S, H), lambda kb: (0, 0, 0)),
            pl.BlockSpec((H, H // 4), lambda kb: (0, 0)),
            pl.BlockSpec((1, H // 4), lambda kb: (0, 0)),
            pl.BlockSpec((1, H // 4), lambda kb: (0, 0)),
            pl.BlockSpec((1, 1), lambda kb: (0, 0)),
            pl.BlockSpec((KB, H), lambda kb: (kb, 0)),
        ],
        out_specs=(
            pl.BlockSpec((B, 128), lambda kb: (0, 0)),
            pl.BlockSpec((1, 128), lambda kb: (0, 0)),
        ),
        out_shape=(
            jax.ShapeDtypeStruct((B, 128), jnp.float32),
            jax.ShapeDtypeStruct((1, 128), jnp.int32),
        ),
        scratch_shapes=[
            pltpu.VMEM((B, H), jnp.float32),
            pltpu.VMEM((B, KB), jnp.float32),
            pltpu.VMEM((B, KB), jnp.int32),
        ],
    )(hidden, g_w1, g_b1.reshape(1, H // 4), g_w2.reshape(1, H // 4),
      g_b2.reshape(1, 1), memory_keys)


def _retrieve(idx, memory_keys):
    mesh = plsc.ScalarSubcoreMesh(axis_name="c", num_cores=1)

    @functools.partial(
        pl.kernel,
        mesh=mesh,
        out_type=jax.ShapeDtypeStruct((B, H), jnp.float32),
        scratch_types=[
            pltpu.SMEM((1, 128), jnp.int32),
            pltpu.SemaphoreType.DMA,
        ],
    )
    def body(idx_hbm, mk_hbm, out_hbm, idx_smem, sem):
        pltpu.sync_copy(idx_hbm, idx_smem)
        cps = []
        for b in range(B):
            i = idx_smem[0, b]
            cps.append(pltpu.async_copy(mk_hbm.at[pl.ds(i, 1)],
                                        out_hbm.at[pl.ds(b, 1)], sem))
        for cp in cps:
            cp.wait()

    return body(idx, memory_keys)


JB = 1


def _fuse_body(fw_ref, h_ref, r_ref, o_ref):
    jb = pl.program_id(0)
    f = fw_ref[:, 0:1]
    for jj in range(JB):
        hh = h_ref[jj]
        rr = r_ref[pl.ds(jb * JB + jj, 1), :]
        d = jnp.broadcast_to(rr, (S, H)) - hh
        for i in range(B):
            o_ref[i, jj] = hh + f[i:i + 1] * d


def _fuse(fw, hidden, retrieved):
    return pl.pallas_call(
        _fuse_body,
        grid=(B // JB,),
        in_specs=[
            pl.BlockSpec((B, 128), lambda j: (0, 0)),
            pl.BlockSpec((JB, S, H), lambda j: (j, 0, 0)),
            pl.BlockSpec((B, H), lambda j: (0, 0)),
        ],
        out_specs=pl.BlockSpec((B, JB, S, H), lambda j: (0, j, 0, 0)),
        out_shape=jax.ShapeDtypeStruct((B, B, S, H), jnp.float32),
    )(fw, hidden, retrieved)


def kernel(hidden_states, memory_keys, g_w1, g_b1, g_w2, g_b2):
    fw, idx = _scores(hidden_states, g_w1, g_b1, g_w2, g_b2, memory_keys)
    retrieved = _retrieve(idx, memory_keys)
    return _fuse(fw, hidden_states, retrieved)


# speedup vs baseline: 1.1520x; 1.1520x over previous
import functools

import jax
import jax.numpy as jnp
from jax.experimental import pallas as pl
from jax.experimental.pallas import tpu as pltpu
from jax.experimental.pallas import tpu_sc as plsc

B, S, H = 8, 512, 768
K_MEM = 65536
KB = 4096
LANES = 16


def _scores_body(h_ref, w1_ref, b1_ref, w2r_ref, b2_ref, mk_ref,
                 fw_ref, idx_ref, q_scr, mv_ref, mt_ref):
    kb = pl.program_id(0)

    @pl.when(kb == 0)
    def _():
        hs = h_ref[...]
        q = jnp.sum(hs, axis=1) * (1.0 / S)
        q_scr[...] = q
        h1 = jnp.maximum(
            jax.lax.dot_general(q, w1_ref[...], (((1,), (0,)), ((), ())),
                                preferred_element_type=jnp.float32)
            + b1_ref[...], 0.0)
        z = jnp.sum(h1 * w2r_ref[...], axis=1, keepdims=True) + b2_ref[...]
        fw_ref[...] = jnp.broadcast_to(jax.nn.sigmoid(z), (B, 128))

    k = mk_ref[...]
    q = q_scr[...]
    qk = jax.lax.dot_general(q, k, (((1,), (1,)), ((), ())),
                             preferred_element_type=jnp.float32)
    kk = k * k
    ksq = jax.lax.dot_general(jnp.ones((B, H), jnp.float32), kk,
                              (((1,), (1,)), ((), ())),
                              preferred_element_type=jnp.float32)
    s = ksq - 2.0 * qk

    @pl.when(kb == 0)
    def _():
        mv_ref[...] = s
        mt_ref[...] = jnp.zeros((B, KB), jnp.int32)

    @pl.when(kb > 0)
    def _():
        old = mv_ref[...]
        p = s < old
        mt_ref[...] = jnp.where(p, kb, mt_ref[...])
        mv_ref[...] = jnp.where(p, s, old)

    @pl.when(kb == K_MEM // KB - 1)
    def _():
        mv = mv_ref[...]
        gm = mt_ref[...] * KB + jax.lax.broadcasted_iota(
            jnp.int32, (B, KB), 1)
        m = jnp.min(mv, axis=1, keepdims=True)
        sel = jnp.where(mv == m, gm, jnp.int32(2**31 - 1))
        idx8 = jnp.min(sel, axis=1)
        idx_ref[...] = jnp.concatenate(
            [idx8[None, :], jnp.zeros((1, 128 - B), jnp.int32)], axis=1)


def _scores(hidden, g_w1, g_b1, g_w2, g_b2, memory_keys):
    return pl.pallas_call(
        _scores_body,
        grid=(K_MEM // KB,),
        in_specs=[
            pl.BlockSpec((B, S, H), lambda kb: (0, 0, 0)),
            pl.BlockSpec((H, H // 4), lambda kb: (0, 0)),
            pl.BlockSpec((1, H // 4), lambda kb: (0, 0)),
            pl.BlockSpec((1, H // 4), lambda kb: (0, 0)),
            pl.BlockSpec((1, 1), lambda kb: (0, 0)),
            pl.BlockSpec((KB, H), lambda kb: (kb, 0)),
        ],
        out_specs=(
            pl.BlockSpec((B, 128), lambda kb: (0, 0)),
            pl.BlockSpec((1, 128), lambda kb: (0, 0)),
        ),
        out_shape=(
            jax.ShapeDtypeStruct((B, 128), jnp.float32),
            jax.ShapeDtypeStruct((1, 128), jnp.int32),
        ),
        scratch_shapes=[
            pltpu.VMEM((B, H), jnp.float32),
            pltpu.VMEM((B, KB), jnp.float32),
            pltpu.VMEM((B, KB), jnp.int32),
        ],
    )(hidden, g_w1, g_b1.reshape(1, H // 4), g_w2.reshape(1, H // 4),
      g_b2.reshape(1, 1), memory_keys)


def _retrieve(idx, memory_keys):
    mesh = plsc.ScalarSubcoreMesh(axis_name="c", num_cores=1)

    @functools.partial(
        pl.kernel,
        mesh=mesh,
        out_type=jax.ShapeDtypeStruct((B, H), jnp.float32),
        scratch_types=[
            pltpu.SMEM((1, 128), jnp.int32),
            pltpu.SemaphoreType.DMA,
        ],
    )
    def body(idx_hbm, mk_hbm, out_hbm, idx_smem, sem):
        pltpu.sync_copy(idx_hbm, idx_smem)
        cps = []
        for b in range(B):
            i = idx_smem[0, b]
            cps.append(pltpu.async_copy(mk_hbm.at[pl.ds(i, 1)],
                                        out_hbm.at[pl.ds(b, 1)], sem))
        for cp in cps:
            cp.wait()

    return body(idx, memory_keys)


JB = 1


def _fuse_body(fw_ref, h_ref, r_ref, o_ref):
    jb = pl.program_id(0)
    f = fw_ref[:, 0:1]
    for jj in range(JB):
        hh = h_ref[jj]
        rr = r_ref[pl.ds(jb * JB + jj, 1), :]
        d = jnp.broadcast_to(rr, (S, H)) - hh
        for i in range(B):
            o_ref[i, jj] = hh + f[i:i + 1] * d


def _fuse(fw, hidden, retrieved):
    return pl.pallas_call(
        _fuse_body,
        grid=(B // JB,),
        in_specs=[
            pl.BlockSpec((B, 128), lambda j: (0, 0)),
            pl.BlockSpec((JB, S, H), lambda j: (j, 0, 0)),
            pl.BlockSpec((B, H), lambda j: (0, 0)),
        ],
        out_specs=pl.BlockSpec((B, JB, S, H), lambda j: (0, j, 0, 0)),
        out_shape=jax.ShapeDtypeStruct((B, B, S, H), jnp.float32),
    )(fw, hidden, retrieved)


def kernel(hidden_states, memory_keys, g_w1, g_b1, g_w2, g_b2):
    fw, idx = _scores(hidden_states, g_w1, g_b1, g_w2, g_b2, memory_keys)
    retrieved = _retrieve(idx, memory_keys)
    return _fuse(fw, hidden_states, retrieved)


# speedup vs baseline: 1.1558x; 1.0033x over previous
import functools

import jax
import jax.numpy as jnp
from jax.experimental import pallas as pl
from jax.experimental.pallas import tpu as pltpu
from jax.experimental.pallas import tpu_sc as plsc

B, S, H = 8, 512, 768
K_MEM = 65536
KB = 4096
LANES = 16


def _scores_body(h_ref, w1_ref, b1_ref, w2r_ref, b2_ref, mk_ref,
                 fw_ref, idx_ref, q_scr, mv_ref, mt_ref):
    kb = pl.program_id(0)

    @pl.when(kb == 0)
    def _():
        hs = h_ref[...]
        q = jnp.sum(hs, axis=1) * (1.0 / S)
        q_scr[...] = q * -2.0
        h1 = jnp.maximum(
            jax.lax.dot_general(q, w1_ref[...], (((1,), (0,)), ((), ())),
                                preferred_element_type=jnp.float32)
            + b1_ref[...], 0.0)
        z = jnp.sum(h1 * w2r_ref[...], axis=1, keepdims=True) + b2_ref[...]
        fw_ref[...] = jnp.broadcast_to(jax.nn.sigmoid(z), (B, 128))

    k = mk_ref[...]
    q2 = q_scr[...]
    qk2 = jax.lax.dot_general(q2, k, (((1,), (1,)), ((), ())),
                              preferred_element_type=jnp.float32)
    kk = k * k
    ksq = jax.lax.dot_general(jnp.ones((B, H), jnp.float32), kk,
                              (((1,), (1,)), ((), ())),
                              preferred_element_type=jnp.float32)
    s = ksq + qk2

    @pl.when(kb == 0)
    def _():
        mv_ref[...] = s
        mt_ref[...] = jnp.zeros((B, KB), jnp.int32)

    @pl.when(kb > 0)
    def _():
        old = mv_ref[...]
        p = s < old
        mt_ref[...] = jnp.where(p, kb, mt_ref[...])
        mv_ref[...] = jnp.where(p, s, old)

    @pl.when(kb == K_MEM // KB - 1)
    def _():
        mv = mv_ref[...]
        gm = mt_ref[...] * KB + jax.lax.broadcasted_iota(
            jnp.int32, (B, KB), 1)
        m = jnp.min(mv, axis=1, keepdims=True)
        sel = jnp.where(mv == m, gm, jnp.int32(2**31 - 1))
        idx8 = jnp.min(sel, axis=1)
        idx_ref[...] = jnp.concatenate(
            [idx8[None, :], jnp.zeros((1, 128 - B), jnp.int32)], axis=1)


def _scores(hidden, g_w1, g_b1, g_w2, g_b2, memory_keys):
    return pl.pallas_call(
        _scores_body,
        grid=(K_MEM // KB,),
        in_specs=[
            pl.BlockSpec((B, S, H), lambda kb: (0, 0, 0)),
            pl.BlockSpec((H, H // 4), lambda kb: (0, 0)),
            pl.BlockSpec((1, H // 4), lambda kb: (0, 0)),
            pl.BlockSpec((1, H // 4), lambda kb: (0, 0)),
            pl.BlockSpec((1, 1), lambda kb: (0, 0)),
            pl.BlockSpec((KB, H), lambda kb: (kb, 0)),
        ],
        out_specs=(
            pl.BlockSpec((B, 128), lambda kb: (0, 0)),
            pl.BlockSpec((1, 128), lambda kb: (0, 0)),
        ),
        out_shape=(
            jax.ShapeDtypeStruct((B, 128), jnp.float32),
            jax.ShapeDtypeStruct((1, 128), jnp.int32),
        ),
        scratch_shapes=[
            pltpu.VMEM((B, H), jnp.float32),
            pltpu.VMEM((B, KB), jnp.float32),
            pltpu.VMEM((B, KB), jnp.int32),
        ],
    )(hidden, g_w1, g_b1.reshape(1, H // 4), g_w2.reshape(1, H // 4),
      g_b2.reshape(1, 1), memory_keys)


def _retrieve(idx, memory_keys):
    mesh = plsc.ScalarSubcoreMesh(axis_name="c", num_cores=1)

    @functools.partial(
        pl.kernel,
        mesh=mesh,
        out_type=jax.ShapeDtypeStruct((B, H), jnp.float32),
        scratch_types=[
            pltpu.SMEM((1, 128), jnp.int32),
            pltpu.SemaphoreType.DMA,
        ],
    )
    def body(idx_hbm, mk_hbm, out_hbm, idx_smem, sem):
        pltpu.sync_copy(idx_hbm, idx_smem)
        cps = []
        for b in range(B):
            i = idx_smem[0, b]
            cps.append(pltpu.async_copy(mk_hbm.at[pl.ds(i, 1)],
                                        out_hbm.at[pl.ds(b, 1)], sem))
        for cp in cps:
            cp.wait()

    return body(idx, memory_keys)


JB = 1


def _fuse_body(fw_ref, h_ref, r_ref, o_ref):
    jb = pl.program_id(0)
    f = fw_ref[:, 0:1]
    for jj in range(JB):
        hh = h_ref[jj]
        rr = r_ref[pl.ds(jb * JB + jj, 1), :]
        d = jnp.broadcast_to(rr, (S, H)) - hh
        for i in range(B):
            o_ref[i, jj] = hh + f[i:i + 1] * d


def _fuse(fw, hidden, retrieved):
    return pl.pallas_call(
        _fuse_body,
        grid=(B // JB,),
        in_specs=[
            pl.BlockSpec((B, 128), lambda j: (0, 0)),
            pl.BlockSpec((JB, S, H), lambda j: (j, 0, 0)),
            pl.BlockSpec((B, H), lambda j: (0, 0)),
        ],
        out_specs=pl.BlockSpec((B, JB, S, H), lambda j: (0, j, 0, 0)),
        out_shape=jax.ShapeDtypeStruct((B, B, S, H), jnp.float32),
    )(fw, hidden, retrieved)


def kernel(hidden_states, memory_keys, g_w1, g_b1, g_w2, g_b2):
    fw, idx = _scores(hidden_states, g_w1, g_b1, g_w2, g_b2, memory_keys)
    retrieved = _retrieve(idx, memory_keys)
    return _fuse(fw, hidden_states, retrieved)


# speedup vs baseline: 1.1897x; 1.0293x over previous
import functools

import jax
import jax.numpy as jnp
from jax.experimental import pallas as pl
from jax.experimental.pallas import tpu as pltpu
from jax.experimental.pallas import tpu_sc as plsc

B, S, H = 8, 512, 768
K_MEM = 65536
KB = 4096


def _scores_body(h_ref, w1_ref, b1_ref, w2r_ref, b2_ref, mk_ref,
                 fw_ref, idx_ref, q_scr, mv_ref, mt_ref):
    kb = pl.program_id(0)

    @pl.when(kb == 0)
    def _():
        hs = h_ref[...]
        q = jnp.sum(hs, axis=1) * (1.0 / S)
        q_scr[...] = q * -2.0
        h1 = jnp.maximum(
            jax.lax.dot_general(q, w1_ref[...], (((1,), (0,)), ((), ())),
                                preferred_element_type=jnp.float32)
            + b1_ref[...], 0.0)
        z = jnp.sum(h1 * w2r_ref[...], axis=1, keepdims=True) + b2_ref[...]
        fw_ref[...] = jnp.broadcast_to(jax.nn.sigmoid(z), (B, 128))

    k = mk_ref[...]
    q2 = q_scr[...]
    qk2 = jax.lax.dot_general(q2, k, (((1,), (1,)), ((), ())),
                              preferred_element_type=jnp.float32)
    kk = k * k
    ksq = jax.lax.dot_general(jnp.ones((B, H), jnp.float32), kk,
                              (((1,), (1,)), ((), ())),
                              preferred_element_type=jnp.float32)
    s = ksq + qk2

    @pl.when(kb == 0)
    def _():
        mv_ref[...] = s
        mt_ref[...] = jnp.zeros((B, KB), jnp.int32)

    @pl.when(kb > 0)
    def _():
        old = mv_ref[...]
        p = s < old
        mt_ref[...] = jnp.where(p, kb, mt_ref[...])
        mv_ref[...] = jnp.where(p, s, old)

    @pl.when(kb == K_MEM // KB - 1)
    def _():
        mv = mv_ref[...]
        gm = mt_ref[...] * KB + jax.lax.broadcasted_iota(
            jnp.int32, (B, KB), 1)
        m = jnp.min(mv, axis=1, keepdims=True)
        sel = jnp.where(mv == m, gm, jnp.int32(2**31 - 1))
        idx8 = jnp.min(sel, axis=1)
        idx_ref[...] = jnp.concatenate(
            [idx8[None, :], jnp.zeros((1, 128 - B), jnp.int32)], axis=1)


def _scores(hidden, g_w1, g_b1, g_w2, g_b2, memory_keys):
    return pl.pallas_call(
        _scores_body,
        grid=(K_MEM // KB,),
        in_specs=[
            pl.BlockSpec((B, S, H), lambda kb: (0, 0, 0)),
            pl.BlockSpec((H, H // 4), lambda kb: (0, 0)),
            pl.BlockSpec((1, H // 4), lambda kb: (0, 0)),
            pl.BlockSpec((1, H // 4), lambda kb: (0, 0)),
            pl.BlockSpec((1, 1), lambda kb: (0, 0)),
            pl.BlockSpec((KB, H), lambda kb: (kb, 0)),
        ],
        out_specs=(
            pl.BlockSpec((B, 128), lambda kb: (0, 0)),
            pl.BlockSpec((1, 128), lambda kb: (0, 0)),
        ),
        out_shape=(
            jax.ShapeDtypeStruct((B, 128), jnp.float32),
            jax.ShapeDtypeStruct((1, 128), jnp.int32),
        ),
        scratch_shapes=[
            pltpu.VMEM((B, H), jnp.float32),
            pltpu.VMEM((B, KB), jnp.float32),
            pltpu.VMEM((B, KB), jnp.int32),
        ],
    )(hidden, g_w1, g_b1.reshape(1, H // 4), g_w2.reshape(1, H // 4),
      g_b2.reshape(1, 1), memory_keys)


def _retrieve(idx, memory_keys):
    mesh = plsc.ScalarSubcoreMesh(axis_name="c", num_cores=1)

    @functools.partial(
        pl.kernel,
        mesh=mesh,
        out_type=jax.ShapeDtypeStruct((B, H), jnp.float32),
        scratch_types=[
            pltpu.SMEM((1, 128), jnp.int32),
            pltpu.SemaphoreType.DMA,
        ],
    )
    def body(idx_hbm, mk_hbm, out_hbm, idx_smem, sem):
        pltpu.sync_copy(idx_hbm, idx_smem)
        cps = []
        for b in range(B):
            i = idx_smem[0, b]
            cps.append(pltpu.async_copy(mk_hbm.at[pl.ds(i, 1)],
                                        out_hbm.at[pl.ds(b, 1)], sem))
        for cp in cps:
            cp.wait()

    return body(idx, memory_keys)


JB = 1


def _fuse_body(fw_ref, h_ref, r_ref, o_ref):
    jb = pl.program_id(0)
    f = fw_ref[:, 0:1]
    for jj in range(JB):
        hh = h_ref[jj]
        rr = r_ref[pl.ds(jb * JB + jj, 1), :]
        d = jnp.broadcast_to(rr, (S, H)) - hh
        for i in range(B):
            o_ref[i, jj] = hh + f[i:i + 1] * d


def _fuse(fw, hidden, retrieved):
    return pl.pallas_call(
        _fuse_body,
        grid=(B // JB,),
        in_specs=[
            pl.BlockSpec((B, 128), lambda j: (0, 0)),
            pl.BlockSpec((JB, S, H), lambda j: (j, 0, 0)),
            pl.BlockSpec((B, H), lambda j: (0, 0)),
        ],
        out_specs=pl.BlockSpec((B, JB, S, H), lambda j: (0, j, 0, 0)),
        out_shape=jax.ShapeDtypeStruct((B, B, S, H), jnp.float32),
    )(fw, hidden, retrieved)


def kernel(hidden_states, memory_keys, g_w1, g_b1, g_w2, g_b2):
    fw, idx = _scores(hidden_states, g_w1, g_b1, g_w2, g_b2, memory_keys)
    retrieved = _retrieve(idx, memory_keys)
    return _fuse(fw, hidden_states, retrieved)
